# baseline (speedup 1.0000x reference)
import functools

import jax
import jax.numpy as jnp
import numpy as np
from jax import lax
from jax.experimental import pallas as pl
from jax.experimental.pallas import tpu as pltpu
from jax.experimental.pallas import tpu_sc as plsc

N = 16384
C = 256
N_BINS = 15
NB = 16
HC = C * NB
OFF_CNT = 0
OFF_CNF = HC
OFF_ACC = 2 * HC
OFF_YT = 3 * HC
HTOT = 3 * HC + 512

NTILES = 16
SPT = N // NTILES
L = 16

_LOWERS = np.linspace(0.0, 1.0, N_BINS + 1)[:-1].astype(np.float32)


_BLK = 8192


def _tc_body(yp_ref, eye_ref, conf_ref, pred_ref):
    x = yp_ref[...]
    m = jnp.max(x, axis=1, keepdims=True)
    e = jnp.exp(x - m)
    ones = jnp.ones((C, 128), jnp.float32)
    s = lax.dot_general(e, ones, (((1,), (0,)), ((), ())),
                        preferred_element_type=jnp.float32)[:, :1]
    conf = 1.0 / s
    iota = lax.broadcasted_iota(jnp.int32, x.shape, 1)
    pred = jnp.min(jnp.where(x == m, iota, jnp.int32(2**30)),
                   axis=1, keepdims=True).astype(jnp.float32)
    cp = jnp.concatenate([conf, pred], axis=1)
    eye = eye_ref[...]
    for k in range(_BLK // 128):
        chunk = cp[k * 128:(k + 1) * 128, :]
        outc = lax.dot_general(chunk, eye, (((0,), (0,)), ((), ())),
                               preferred_element_type=jnp.float32)
        conf_ref[0, 0, pl.ds(k * 128, 128)] = outc[0]
        pred_ref[0, 0, pl.ds(k * 128, 128)] = outc[1]


def _tc_stage(y_pred):
    g = N // _BLK
    conf, pred = pl.pallas_call(
        _tc_body,
        grid=(g,),
        in_specs=[
            pl.BlockSpec((_BLK, C), lambda i: (i, 0)),
            pl.BlockSpec((128, 128), lambda i: (0, 0)),
        ],
        out_specs=[
            pl.BlockSpec((1, 1, _BLK), lambda i: (i, 0, 0)),
            pl.BlockSpec((1, 1, _BLK), lambda i: (i, 0, 0)),
        ],
        out_shape=[
            jax.ShapeDtypeStruct((g, 1, _BLK), jnp.float32),
            jax.ShapeDtypeStruct((g, 1, _BLK), jnp.float32),
        ],
    )(y_pred, jnp.eye(128, dtype=jnp.float32))
    return conf.reshape(N), pred.reshape(N)


def _sc_body(cnf_hbm, prd_hbm, yt_hbm, zeros_hbm, out_hbm,
             cnf_v, prd_v, yt_v, hist_v,
             blkc, blkf, blka, blkyt, redc, redf, reda, redyt,
             cnt16, cnf16, acc16, yt16, stage16, stage16b,
             sh_all, sh_cnt, sh_cnf, sh_acc, sh_yt, sh_p, sh_c, sem):
    cid = lax.axis_index("c")
    wid = lax.axis_index("s")

    def batch(*copies):
        handles = [pltpu.async_copy(s, d, sem) for s, d in copies]
        for h in handles:
            h.wait()

    @pl.when(cid == 0)
    def _():
        ones = jnp.full((L,), 1.0, jnp.float32)

        base = wid * SPT
        batch((zeros_hbm, hist_v),
              (cnf_hbm.at[pl.ds(base, SPT)], cnf_v),
              (prd_hbm.at[pl.ds(base, SPT)], prd_v),
              (yt_hbm.at[pl.ds(base, SPT)], yt_v))
        def scatter_body(i, _):
            c16 = cnf_v[pl.ds(i * L, L)]
            p16 = prd_v[pl.ds(i * L, L)].astype(jnp.int32)
            t16 = yt_v[pl.ds(i * L, L)]
            a16 = (p16 == t16).astype(jnp.float32)
            t = c16 * jnp.float32(N_BINS)
            f = t.astype(jnp.int32)
            b16 = f - (t == f.astype(jnp.float32)).astype(jnp.int32)
            k16 = b16 * C + p16
            plsc.addupdate_scatter(hist_v, [k16], ones)
            plsc.addupdate_scatter(hist_v, [k16 + OFF_CNF], c16)
            plsc.addupdate_scatter(hist_v, [k16 + OFF_ACC], a16)
            plsc.addupdate_scatter(hist_v, [t16 + OFF_YT], ones)
            return 0

        lax.fori_loop(0, SPT // L, scatter_body, 0)

        pltpu.sync_copy(hist_v, sh_all.at[wid])
        plsc.subcore_barrier()

        batch((sh_all.at[:, pl.ds(OFF_CNT + wid * 256, 256)], blkc),
              (sh_all.at[:, pl.ds(OFF_CNF + wid * 256, 256)], blkf),
              (sh_all.at[:, pl.ds(OFF_ACC + wid * 256, 256)], blka),
              (sh_all.at[:, pl.ds(OFF_YT + wid * 32, 32)], blkyt))

        def reduce_region(blk, red, width):
            def body(j, _):
                sl = pl.ds(j * L, L)
                acc = blk[0, sl]
                for t in range(1, NTILES):
                    acc = acc + blk[t, sl]
                red[sl] = acc
                return 0
            lax.fori_loop(0, width // L, body, 0)

        reduce_region(blkc, redc, 256)
        reduce_region(blkf, redf, 256)
        reduce_region(blka, reda, 256)
        reduce_region(blkyt, redyt, 32)
        batch((redc, sh_cnt.at[wid]),
              (redf, sh_cnf.at[wid]),
              (reda, sh_acc.at[wid]),
              (redyt, sh_yt.at[pl.ds(wid * 32, 32)]))
        plsc.subcore_barrier()

        cls0 = wid * L
        batch((sh_cnt.at[:, pl.ds(cls0, L)], cnt16),
              (sh_cnf.at[:, pl.ds(cls0, L)], cnf16),
              (sh_acc.at[:, pl.ds(cls0, L)], acc16),
              (sh_yt.at[pl.ds(cls0, L)], yt16))
        n_c = jnp.zeros((L,), jnp.float32)
        esum = jnp.zeros((L,), jnp.float32)
        for b in range(NB):
            kv = cnt16[b]
            cv = cnf16[b]
            av = acc16[b]
            n_c = n_c + kv
            esum = esum + jnp.where(kv > 0.0, jnp.abs(cv - av), 0.0)
        ece = esum / jnp.maximum(n_c, 1.0)
        pv = (yt16[...] > 0.0).astype(jnp.float32)
        stage16[...] = jnp.broadcast_to(jnp.sum(pv * ece), (L,))
        stage16b[...] = jnp.broadcast_to(jnp.sum(pv), (L,))
        batch((stage16, sh_p.at[wid]),
              (stage16b, sh_c.at[wid]))
        plsc.subcore_barrier()

        @pl.when(wid == 0)
        def _():
            batch((sh_p, cnt16),
                  (sh_c, cnf16))
            accp = cnt16[0]
            accc = cnf16[0]
            for t in range(1, NTILES):
                accp = accp + cnt16[t]
                accc = accc + cnf16[t]
            stage16[...] = accp / accc
            pltpu.sync_copy(stage16, out_hbm)


@functools.lru_cache(maxsize=1)
def _sc_stage_fn():
    return pl.kernel(
        _sc_body,
        mesh=plsc.VectorSubcoreMesh(core_axis_name="c", subcore_axis_name="s", num_cores=1),
        compiler_params=pltpu.CompilerParams(
            use_tc_tiling_on_sc=False, needs_layout_passes=False),
        out_type=jax.ShapeDtypeStruct((L,), jnp.float32),
        scratch_types=[
        pltpu.VMEM((SPT,), jnp.float32),
        pltpu.VMEM((SPT,), jnp.float32),
        pltpu.VMEM((SPT,), jnp.int32),
        pltpu.VMEM((HTOT,), jnp.float32),
        pltpu.VMEM((NTILES, 256), jnp.float32),
        pltpu.VMEM((NTILES, 256), jnp.float32),
        pltpu.VMEM((NTILES, 256), jnp.float32),
        pltpu.VMEM((NTILES, 32), jnp.float32),
        pltpu.VMEM((256,), jnp.float32),
        pltpu.VMEM((256,), jnp.float32),
        pltpu.VMEM((256,), jnp.float32),
        pltpu.VMEM((32,), jnp.float32),
        pltpu.VMEM((NB, L), jnp.float32),
        pltpu.VMEM((NB, L), jnp.float32),
        pltpu.VMEM((NB, L), jnp.float32),
        pltpu.VMEM((L,), jnp.float32),
        pltpu.VMEM((L,), jnp.float32),
        pltpu.VMEM((L,), jnp.float32),
        pltpu.VMEM_SHARED((NTILES, HTOT), jnp.float32),
        pltpu.VMEM_SHARED((NB, C), jnp.float32),
        pltpu.VMEM_SHARED((NB, C), jnp.float32),
        pltpu.VMEM_SHARED((NB, C), jnp.float32),
        pltpu.VMEM_SHARED((512,), jnp.float32),
        pltpu.VMEM_SHARED((NTILES, L), jnp.float32),
        pltpu.VMEM_SHARED((NTILES, L), jnp.float32),
        pltpu.SemaphoreType.DMA,
        ],
    )


def kernel(y_pred, y_true):
    conf, pred = _tc_stage(y_pred)
    zeros = jnp.zeros((HTOT,), jnp.float32)
    out = _sc_stage_fn()(conf, pred, y_true, zeros)
    return out[0:1]


# speedup vs baseline: 1.0318x; 1.0318x over previous
import functools

import jax
import jax.numpy as jnp
import numpy as np
from jax import lax
from jax.experimental import pallas as pl
from jax.experimental.pallas import tpu as pltpu
from jax.experimental.pallas import tpu_sc as plsc

N = 16384
C = 256
N_BINS = 15
NB = 16
HC = C * NB
OFF_CNT = 0
OFF_CNF = HC
OFF_ACC = 2 * HC
OFF_YT = 3 * HC
HTOT = 3 * HC + 512

NTILES = 16
SPT = N // NTILES
L = 16

_LOWERS = np.linspace(0.0, 1.0, N_BINS + 1)[:-1].astype(np.float32)


_BLK = 4096


def _tc_body(yp_ref, eye_ref, conf_ref, pred_ref):
    x = yp_ref[...]
    m = jnp.max(x, axis=1, keepdims=True)
    e = jnp.exp(x - m)
    ones = jnp.ones((C, 128), jnp.float32)
    s = lax.dot_general(e, ones, (((1,), (0,)), ((), ())),
                        preferred_element_type=jnp.float32)[:, :1]
    conf = 1.0 / s
    iota = lax.broadcasted_iota(jnp.int32, x.shape, 1)
    pred = jnp.min(jnp.where(x == m, iota, jnp.int32(2**30)),
                   axis=1, keepdims=True).astype(jnp.float32)
    cp = jnp.concatenate([conf, pred], axis=1)
    eye = eye_ref[...]
    for k in range(_BLK // 128):
        chunk = cp[k * 128:(k + 1) * 128, :]
        outc = lax.dot_general(chunk, eye, (((0,), (0,)), ((), ())),
                               preferred_element_type=jnp.float32)
        conf_ref[0, 0, pl.ds(k * 128, 128)] = outc[0]
        pred_ref[0, 0, pl.ds(k * 128, 128)] = outc[1]


def _tc_stage(y_pred):
    g = N // _BLK
    conf, pred = pl.pallas_call(
        _tc_body,
        grid=(g,),
        in_specs=[
            pl.BlockSpec((_BLK, C), lambda i: (i, 0)),
            pl.BlockSpec((128, 128), lambda i: (0, 0)),
        ],
        out_specs=[
            pl.BlockSpec((1, 1, _BLK), lambda i: (i, 0, 0)),
            pl.BlockSpec((1, 1, _BLK), lambda i: (i, 0, 0)),
        ],
        out_shape=[
            jax.ShapeDtypeStruct((g, 1, _BLK), jnp.float32),
            jax.ShapeDtypeStruct((g, 1, _BLK), jnp.float32),
        ],
    )(y_pred, jnp.eye(128, dtype=jnp.float32))
    return conf.reshape(N), pred.reshape(N)


def _sc_body(cnf_hbm, prd_hbm, yt_hbm, zeros_hbm, out_hbm,
             cnf_v, prd_v, yt_v, hist_v,
             blkc, blkf, blka, blkyt, redc, redf, reda, redyt,
             cnt16, cnf16, acc16, yt16, stage16, stage16b,
             sh_all, sh_cnt, sh_cnf, sh_acc, sh_yt, sh_p, sh_c, sem):
    cid = lax.axis_index("c")
    wid = lax.axis_index("s")

    def batch(*copies):
        handles = [pltpu.async_copy(s, d, sem) for s, d in copies]
        for h in handles:
            h.wait()

    @pl.when(cid == 0)
    def _():
        ones = jnp.full((L,), 1.0, jnp.float32)

        base = wid * SPT
        batch((zeros_hbm, hist_v),
              (cnf_hbm.at[pl.ds(base, SPT)], cnf_v),
              (prd_hbm.at[pl.ds(base, SPT)], prd_v),
              (yt_hbm.at[pl.ds(base, SPT)], yt_v))
        def scatter_body(i, _):
            for u in range(4):
                sl = pl.ds(i * (4 * L) + u * L, L)
                c16 = cnf_v[sl]
                p16 = prd_v[sl].astype(jnp.int32)
                t16 = yt_v[sl]
                a16 = (p16 == t16).astype(jnp.float32)
                t = c16 * jnp.float32(N_BINS)
                f = t.astype(jnp.int32)
                b16 = f - (t == f.astype(jnp.float32)).astype(jnp.int32)
                k16 = b16 * C + p16
                plsc.addupdate_scatter(hist_v, [k16], ones)
                plsc.addupdate_scatter(hist_v, [k16 + OFF_CNF], c16)
                plsc.addupdate_scatter(hist_v, [k16 + OFF_ACC], a16)
                plsc.addupdate_scatter(hist_v, [t16 + OFF_YT], ones)
            return 0

        lax.fori_loop(0, SPT // L // 4, scatter_body, 0)

        pltpu.sync_copy(hist_v, sh_all.at[wid])
        plsc.subcore_barrier()

        batch((sh_all.at[:, pl.ds(OFF_CNT + wid * 256, 256)], blkc),
              (sh_all.at[:, pl.ds(OFF_CNF + wid * 256, 256)], blkf),
              (sh_all.at[:, pl.ds(OFF_ACC + wid * 256, 256)], blka),
              (sh_all.at[:, pl.ds(OFF_YT + wid * 32, 32)], blkyt))

        def reduce_region(blk, red, width):
            def body(j, _):
                sl = pl.ds(j * L, L)
                acc = blk[0, sl]
                for t in range(1, NTILES):
                    acc = acc + blk[t, sl]
                red[sl] = acc
                return 0
            lax.fori_loop(0, width // L, body, 0)

        reduce_region(blkc, redc, 256)
        reduce_region(blkf, redf, 256)
        reduce_region(blka, reda, 256)
        reduce_region(blkyt, redyt, 32)
        batch((redc, sh_cnt.at[wid]),
              (redf, sh_cnf.at[wid]),
              (reda, sh_acc.at[wid]),
              (redyt, sh_yt.at[pl.ds(wid * 32, 32)]))
        plsc.subcore_barrier()

        cls0 = wid * L
        batch((sh_cnt.at[:, pl.ds(cls0, L)], cnt16),
              (sh_cnf.at[:, pl.ds(cls0, L)], cnf16),
              (sh_acc.at[:, pl.ds(cls0, L)], acc16),
              (sh_yt.at[pl.ds(cls0, L)], yt16))
        n_c = jnp.zeros((L,), jnp.float32)
        esum = jnp.zeros((L,), jnp.float32)
        for b in range(NB):
            kv = cnt16[b]
            cv = cnf16[b]
            av = acc16[b]
            n_c = n_c + kv
            esum = esum + jnp.where(kv > 0.0, jnp.abs(cv - av), 0.0)
        ece = esum / jnp.maximum(n_c, 1.0)
        pv = (yt16[...] > 0.0).astype(jnp.float32)
        stage16[...] = jnp.broadcast_to(jnp.sum(pv * ece), (L,))
        stage16b[...] = jnp.broadcast_to(jnp.sum(pv), (L,))
        batch((stage16, sh_p.at[wid]),
              (stage16b, sh_c.at[wid]))
        plsc.subcore_barrier()

        @pl.when(wid == 0)
        def _():
            batch((sh_p, cnt16),
                  (sh_c, cnf16))
            accp = cnt16[0]
            accc = cnf16[0]
            for t in range(1, NTILES):
                accp = accp + cnt16[t]
                accc = accc + cnf16[t]
            stage16[...] = accp / accc
            pltpu.sync_copy(stage16, out_hbm)


@functools.lru_cache(maxsize=1)
def _sc_stage_fn():
    return pl.kernel(
        _sc_body,
        mesh=plsc.VectorSubcoreMesh(core_axis_name="c", subcore_axis_name="s", num_cores=1),
        compiler_params=pltpu.CompilerParams(
            use_tc_tiling_on_sc=False, needs_layout_passes=False),
        out_type=jax.ShapeDtypeStruct((L,), jnp.float32),
        scratch_types=[
        pltpu.VMEM((SPT,), jnp.float32),
        pltpu.VMEM((SPT,), jnp.float32),
        pltpu.VMEM((SPT,), jnp.int32),
        pltpu.VMEM((HTOT,), jnp.float32),
        pltpu.VMEM((NTILES, 256), jnp.float32),
        pltpu.VMEM((NTILES, 256), jnp.float32),
        pltpu.VMEM((NTILES, 256), jnp.float32),
        pltpu.VMEM((NTILES, 32), jnp.float32),
        pltpu.VMEM((256,), jnp.float32),
        pltpu.VMEM((256,), jnp.float32),
        pltpu.VMEM((256,), jnp.float32),
        pltpu.VMEM((32,), jnp.float32),
        pltpu.VMEM((NB, L), jnp.float32),
        pltpu.VMEM((NB, L), jnp.float32),
        pltpu.VMEM((NB, L), jnp.float32),
        pltpu.VMEM((L,), jnp.float32),
        pltpu.VMEM((L,), jnp.float32),
        pltpu.VMEM((L,), jnp.float32),
        pltpu.VMEM_SHARED((NTILES, HTOT), jnp.float32),
        pltpu.VMEM_SHARED((NB, C), jnp.float32),
        pltpu.VMEM_SHARED((NB, C), jnp.float32),
        pltpu.VMEM_SHARED((NB, C), jnp.float32),
        pltpu.VMEM_SHARED((512,), jnp.float32),
        pltpu.VMEM_SHARED((NTILES, L), jnp.float32),
        pltpu.VMEM_SHARED((NTILES, L), jnp.float32),
        pltpu.SemaphoreType.DMA,
        ],
    )


def kernel(y_pred, y_true):
    conf, pred = _tc_stage(y_pred)
    zeros = jnp.zeros((HTOT,), jnp.float32)
    out = _sc_stage_fn()(conf, pred, y_true, zeros)
    return out[0:1]


# speedup vs baseline: 1.0829x; 1.0495x over previous
import functools

import jax
import jax.numpy as jnp
import numpy as np
from jax import lax
from jax.experimental import pallas as pl
from jax.experimental.pallas import tpu as pltpu
from jax.experimental.pallas import tpu_sc as plsc

N = 16384
C = 256
N_BINS = 15
NB = 16
HC = C * NB
OFF_CNT = 0
OFF_CNF = HC
OFF_ACC = 2 * HC
OFF_YT = 3 * HC
HTOT = 3 * HC + 512

NTILES = 16
SPT = N // NTILES
L = 16

_LOWERS = np.linspace(0.0, 1.0, N_BINS + 1)[:-1].astype(np.float32)


_BLK = 4096


def _tc_body(yp_ref, conf_ref, pred_ref):
    x = yp_ref[...]
    m = jnp.max(x, axis=1, keepdims=True)
    e = jnp.exp(x - m)
    ones = jnp.ones((C, 128), jnp.float32)
    s = lax.dot_general(e, ones, (((1,), (0,)), ((), ())),
                        preferred_element_type=jnp.float32)[:, :1]
    conf = 1.0 / s
    iota = lax.broadcasted_iota(jnp.int32, x.shape, 1)
    pred = jnp.min(jnp.where(x == m, iota, jnp.int32(2**30)),
                   axis=1, keepdims=True).astype(jnp.float32)
    cp = jnp.concatenate([conf, pred], axis=1)
    r0 = lax.broadcasted_iota(jnp.int32, (128, 128), 0)
    r1 = lax.broadcasted_iota(jnp.int32, (128, 128), 1)
    eye = (r0 == r1).astype(jnp.float32)
    for k in range(_BLK // 128):
        chunk = cp[k * 128:(k + 1) * 128, :]
        outc = lax.dot_general(chunk, eye, (((0,), (0,)), ((), ())),
                               preferred_element_type=jnp.float32)
        conf_ref[0, 0, pl.ds(k * 128, 128)] = outc[0]
        pred_ref[0, 0, pl.ds(k * 128, 128)] = outc[1]


def _tc_stage(y_pred):
    g = N // _BLK
    conf, pred = pl.pallas_call(
        _tc_body,
        grid=(g,),
        in_specs=[
            pl.BlockSpec((_BLK, C), lambda i: (i, 0)),
        ],
        out_specs=[
            pl.BlockSpec((1, 1, _BLK), lambda i: (i, 0, 0)),
            pl.BlockSpec((1, 1, _BLK), lambda i: (i, 0, 0)),
        ],
        out_shape=[
            jax.ShapeDtypeStruct((g, 1, _BLK), jnp.float32),
            jax.ShapeDtypeStruct((g, 1, _BLK), jnp.float32),
        ],
    )(y_pred)
    return conf.reshape(N), pred.reshape(N)


def _sc_body(cnf_hbm, prd_hbm, yt_hbm, out_hbm,
             cnf_v, prd_v, yt_v, hist_v,
             blkc, blkf, blka, blkyt, redc, redf, reda, redyt,
             cnt16, cnf16, acc16, yt16, stage16, stage16b,
             sh_all, sh_cnt, sh_cnf, sh_acc, sh_yt, sh_p, sh_c, sem):
    cid = lax.axis_index("c")
    wid = lax.axis_index("s")

    def batch(*copies):
        handles = [pltpu.async_copy(s, d, sem) for s, d in copies]
        for h in handles:
            h.wait()

    @pl.when(cid == 0)
    def _():
        ones = jnp.full((L,), 1.0, jnp.float32)

        base = wid * SPT
        batch((cnf_hbm.at[pl.ds(base, SPT)], cnf_v),
              (prd_hbm.at[pl.ds(base, SPT)], prd_v),
              (yt_hbm.at[pl.ds(base, SPT)], yt_v))
        zero = jnp.zeros((L,), jnp.float32)

        def zero_body(i, _):
            for u in range(4):
                hist_v[pl.ds(i * (4 * L) + u * L, L)] = zero
            return 0

        lax.fori_loop(0, HTOT // L // 4, zero_body, 0)
        def scatter_body(i, _):
            for u in range(4):
                sl = pl.ds(i * (4 * L) + u * L, L)
                c16 = cnf_v[sl]
                p16 = prd_v[sl].astype(jnp.int32)
                t16 = yt_v[sl]
                a16 = (p16 == t16).astype(jnp.float32)
                t = c16 * jnp.float32(N_BINS)
                f = t.astype(jnp.int32)
                b16 = f - (t == f.astype(jnp.float32)).astype(jnp.int32)
                k16 = b16 * C + p16
                plsc.addupdate_scatter(hist_v, [k16], ones)
                plsc.addupdate_scatter(hist_v, [k16 + OFF_CNF], c16)
                plsc.addupdate_scatter(hist_v, [k16 + OFF_ACC], a16)
                plsc.addupdate_scatter(hist_v, [t16 + OFF_YT], ones)
            return 0

        lax.fori_loop(0, SPT // L // 4, scatter_body, 0)

        pltpu.sync_copy(hist_v, sh_all.at[wid])
        plsc.subcore_barrier()

        batch((sh_all.at[:, pl.ds(OFF_CNT + wid * 256, 256)], blkc),
              (sh_all.at[:, pl.ds(OFF_CNF + wid * 256, 256)], blkf),
              (sh_all.at[:, pl.ds(OFF_ACC + wid * 256, 256)], blka),
              (sh_all.at[:, pl.ds(OFF_YT + wid * 32, 32)], blkyt))

        def reduce_region(blk, red, width):
            def body(j, _):
                sl = pl.ds(j * L, L)
                acc = blk[0, sl]
                for t in range(1, NTILES):
                    acc = acc + blk[t, sl]
                red[sl] = acc
                return 0
            lax.fori_loop(0, width // L, body, 0)

        reduce_region(blkc, redc, 256)
        reduce_region(blkf, redf, 256)
        reduce_region(blka, reda, 256)
        reduce_region(blkyt, redyt, 32)
        batch((redc, sh_cnt.at[wid]),
              (redf, sh_cnf.at[wid]),
              (reda, sh_acc.at[wid]),
              (redyt, sh_yt.at[pl.ds(wid * 32, 32)]))
        plsc.subcore_barrier()

        cls0 = wid * L
        batch((sh_cnt.at[:, pl.ds(cls0, L)], cnt16),
              (sh_cnf.at[:, pl.ds(cls0, L)], cnf16),
              (sh_acc.at[:, pl.ds(cls0, L)], acc16),
              (sh_yt.at[pl.ds(cls0, L)], yt16))
        n_c = jnp.zeros((L,), jnp.float32)
        esum = jnp.zeros((L,), jnp.float32)
        for b in range(NB):
            kv = cnt16[b]
            cv = cnf16[b]
            av = acc16[b]
            n_c = n_c + kv
            esum = esum + jnp.where(kv > 0.0, jnp.abs(cv - av), 0.0)
        ece = esum / jnp.maximum(n_c, 1.0)
        pv = (yt16[...] > 0.0).astype(jnp.float32)
        stage16[...] = jnp.broadcast_to(jnp.sum(pv * ece), (L,))
        stage16b[...] = jnp.broadcast_to(jnp.sum(pv), (L,))
        batch((stage16, sh_p.at[wid]),
              (stage16b, sh_c.at[wid]))
        plsc.subcore_barrier()

        @pl.when(wid == 0)
        def _():
            batch((sh_p, cnt16),
                  (sh_c, cnf16))
            accp = cnt16[0]
            accc = cnf16[0]
            for t in range(1, NTILES):
                accp = accp + cnt16[t]
                accc = accc + cnf16[t]
            stage16[...] = accp / accc
            pltpu.sync_copy(stage16, out_hbm)


@functools.lru_cache(maxsize=1)
def _sc_stage_fn():
    return pl.kernel(
        _sc_body,
        mesh=plsc.VectorSubcoreMesh(core_axis_name="c", subcore_axis_name="s", num_cores=1),
        compiler_params=pltpu.CompilerParams(
            use_tc_tiling_on_sc=False, needs_layout_passes=False,
            skip_device_barrier=True),
        out_type=jax.ShapeDtypeStruct((L,), jnp.float32),
        scratch_types=[
        pltpu.VMEM((SPT,), jnp.float32),
        pltpu.VMEM((SPT,), jnp.float32),
        pltpu.VMEM((SPT,), jnp.int32),
        pltpu.VMEM((HTOT,), jnp.float32),
        pltpu.VMEM((NTILES, 256), jnp.float32),
        pltpu.VMEM((NTILES, 256), jnp.float32),
        pltpu.VMEM((NTILES, 256), jnp.float32),
        pltpu.VMEM((NTILES, 32), jnp.float32),
        pltpu.VMEM((256,), jnp.float32),
        pltpu.VMEM((256,), jnp.float32),
        pltpu.VMEM((256,), jnp.float32),
        pltpu.VMEM((32,), jnp.float32),
        pltpu.VMEM((NB, L), jnp.float32),
        pltpu.VMEM((NB, L), jnp.float32),
        pltpu.VMEM((NB, L), jnp.float32),
        pltpu.VMEM((L,), jnp.float32),
        pltpu.VMEM((L,), jnp.float32),
        pltpu.VMEM((L,), jnp.float32),
        pltpu.VMEM_SHARED((NTILES, HTOT), jnp.float32),
        pltpu.VMEM_SHARED((NB, C), jnp.float32),
        pltpu.VMEM_SHARED((NB, C), jnp.float32),
        pltpu.VMEM_SHARED((NB, C), jnp.float32),
        pltpu.VMEM_SHARED((512,), jnp.float32),
        pltpu.VMEM_SHARED((NTILES, L), jnp.float32),
        pltpu.VMEM_SHARED((NTILES, L), jnp.float32),
        pltpu.SemaphoreType.DMA,
        ],
    )


def kernel(y_pred, y_true):
    conf, pred = _tc_stage(y_pred)
    out = _sc_stage_fn()(conf, pred, y_true)
    return out[0:1]


# speedup vs baseline: 1.0836x; 1.0007x over previous
import functools

import jax
import jax.numpy as jnp
import numpy as np
from jax import lax
from jax.experimental import pallas as pl
from jax.experimental.pallas import tpu as pltpu
from jax.experimental.pallas import tpu_sc as plsc

N = 16384
C = 256
N_BINS = 15
NB = 16
HC = C * NB
OFF_CNT = 0
OFF_CNF = HC
OFF_ACC = 2 * HC
OFF_YT = 3 * HC
HTOT = 3 * HC + 512

NTILES = 16
SPT = N // NTILES
L = 16

_LOWERS = np.linspace(0.0, 1.0, N_BINS + 1)[:-1].astype(np.float32)


_BLK = 4096


def _tc_body(yp_ref, conf_ref, pred_ref):
    x = yp_ref[...]
    m = jnp.max(x, axis=1, keepdims=True)
    e = jnp.exp(x - m)
    ones = jnp.ones((C, 128), jnp.float32)
    s = lax.dot_general(e, ones, (((1,), (0,)), ((), ())),
                        preferred_element_type=jnp.float32)[:, :1]
    conf = 1.0 / s
    iota = lax.broadcasted_iota(jnp.int32, x.shape, 1)
    pred = jnp.min(jnp.where(x == m, iota, jnp.int32(2**30)),
                   axis=1, keepdims=True).astype(jnp.float32)
    cp = jnp.concatenate([conf, pred], axis=1)
    r0 = lax.broadcasted_iota(jnp.int32, (128, 128), 0)
    r1 = lax.broadcasted_iota(jnp.int32, (128, 128), 1)
    eye = (r0 == r1).astype(jnp.float32)
    for k in range(_BLK // 128):
        chunk = cp[k * 128:(k + 1) * 128, :]
        outc = lax.dot_general(chunk, eye, (((0,), (0,)), ((), ())),
                               preferred_element_type=jnp.float32)
        conf_ref[0, 0, pl.ds(k * 128, 128)] = outc[0]
        pred_ref[0, 0, pl.ds(k * 128, 128)] = outc[1]


def _tc_stage(y_pred):
    g = N // _BLK
    conf, pred = pl.pallas_call(
        _tc_body,
        grid=(g,),
        in_specs=[
            pl.BlockSpec((_BLK, C), lambda i: (i, 0)),
        ],
        out_specs=[
            pl.BlockSpec((1, 1, _BLK), lambda i: (i, 0, 0)),
            pl.BlockSpec((1, 1, _BLK), lambda i: (i, 0, 0)),
        ],
        out_shape=[
            jax.ShapeDtypeStruct((g, 1, _BLK), jnp.float32),
            jax.ShapeDtypeStruct((g, 1, _BLK), jnp.float32),
        ],
        compiler_params=pltpu.CompilerParams(skip_device_barrier=True),
    )(y_pred)
    return conf.reshape(N), pred.reshape(N)


def _sc_body(cnf_hbm, prd_hbm, yt_hbm, out_hbm,
             cnf_v, prd_v, yt_v, hist_v,
             blkc, blkf, blka, blkyt, redc, redf, reda, redyt,
             cnt16, cnf16, acc16, yt16, stage16, stage16b,
             sh_all, sh_cnt, sh_cnf, sh_acc, sh_yt, sh_p, sh_c, sem):
    cid = lax.axis_index("c")
    wid = lax.axis_index("s")

    def batch(*copies):
        handles = [pltpu.async_copy(s, d, sem) for s, d in copies]
        for h in handles:
            h.wait()

    @pl.when(cid == 0)
    def _():
        ones = jnp.full((L,), 1.0, jnp.float32)

        base = wid * SPT
        batch((cnf_hbm.at[pl.ds(base, SPT)], cnf_v),
              (prd_hbm.at[pl.ds(base, SPT)], prd_v),
              (yt_hbm.at[pl.ds(base, SPT)], yt_v))
        zero = jnp.zeros((L,), jnp.float32)

        def zero_body(i, _):
            for u in range(4):
                hist_v[pl.ds(i * (4 * L) + u * L, L)] = zero
            return 0

        lax.fori_loop(0, HTOT // L // 4, zero_body, 0)
        def scatter_body(i, _):
            for u in range(4):
                sl = pl.ds(i * (4 * L) + u * L, L)
                c16 = cnf_v[sl]
                p16 = prd_v[sl].astype(jnp.int32)
                t16 = yt_v[sl]
                a16 = (p16 == t16).astype(jnp.float32)
                t = c16 * jnp.float32(N_BINS)
                f = t.astype(jnp.int32)
                b16 = f - (t == f.astype(jnp.float32)).astype(jnp.int32)
                k16 = b16 * C + p16
                plsc.addupdate_scatter(hist_v, [k16], ones)
                plsc.addupdate_scatter(hist_v, [k16 + OFF_CNF], c16)
                plsc.addupdate_scatter(hist_v, [k16 + OFF_ACC], a16)
                plsc.addupdate_scatter(hist_v, [t16 + OFF_YT], ones)
            return 0

        lax.fori_loop(0, SPT // L // 4, scatter_body, 0)

        pltpu.sync_copy(hist_v, sh_all.at[wid])
        plsc.subcore_barrier()

        batch((sh_all.at[:, pl.ds(OFF_CNT + wid * 256, 256)], blkc),
              (sh_all.at[:, pl.ds(OFF_CNF + wid * 256, 256)], blkf),
              (sh_all.at[:, pl.ds(OFF_ACC + wid * 256, 256)], blka),
              (sh_all.at[:, pl.ds(OFF_YT + wid * 32, 32)], blkyt))

        def reduce_region(blk, red, width):
            def body(j, _):
                sl = pl.ds(j * L, L)
                acc = blk[0, sl]
                for t in range(1, NTILES):
                    acc = acc + blk[t, sl]
                red[sl] = acc
                return 0
            lax.fori_loop(0, width // L, body, 0)

        reduce_region(blkc, redc, 256)
        reduce_region(blkf, redf, 256)
        reduce_region(blka, reda, 256)
        reduce_region(blkyt, redyt, 32)
        batch((redc, sh_cnt.at[wid]),
              (redf, sh_cnf.at[wid]),
              (reda, sh_acc.at[wid]),
              (redyt, sh_yt.at[pl.ds(wid * 32, 32)]))
        plsc.subcore_barrier()

        cls0 = wid * L
        batch((sh_cnt.at[:, pl.ds(cls0, L)], cnt16),
              (sh_cnf.at[:, pl.ds(cls0, L)], cnf16),
              (sh_acc.at[:, pl.ds(cls0, L)], acc16),
              (sh_yt.at[pl.ds(cls0, L)], yt16))
        n_c = jnp.zeros((L,), jnp.float32)
        esum = jnp.zeros((L,), jnp.float32)
        for b in range(NB):
            kv = cnt16[b]
            cv = cnf16[b]
            av = acc16[b]
            n_c = n_c + kv
            esum = esum + jnp.where(kv > 0.0, jnp.abs(cv - av), 0.0)
        ece = esum / jnp.maximum(n_c, 1.0)
        pv = (yt16[...] > 0.0).astype(jnp.float32)
        stage16[...] = jnp.broadcast_to(jnp.sum(pv * ece), (L,))
        stage16b[...] = jnp.broadcast_to(jnp.sum(pv), (L,))
        batch((stage16, sh_p.at[wid]),
              (stage16b, sh_c.at[wid]))
        plsc.subcore_barrier()

        @pl.when(wid == 0)
        def _():
            batch((sh_p, cnt16),
                  (sh_c, cnf16))
            accp = cnt16[0]
            accc = cnf16[0]
            for t in range(1, NTILES):
                accp = accp + cnt16[t]
                accc = accc + cnf16[t]
            stage16[...] = accp / accc
            pltpu.sync_copy(stage16, out_hbm)


@functools.lru_cache(maxsize=1)
def _sc_stage_fn():
    return pl.kernel(
        _sc_body,
        mesh=plsc.VectorSubcoreMesh(core_axis_name="c", subcore_axis_name="s", num_cores=1),
        compiler_params=pltpu.CompilerParams(
            use_tc_tiling_on_sc=False, needs_layout_passes=False,
            skip_device_barrier=True),
        out_type=jax.ShapeDtypeStruct((L,), jnp.float32),
        scratch_types=[
        pltpu.VMEM((SPT,), jnp.float32),
        pltpu.VMEM((SPT,), jnp.float32),
        pltpu.VMEM((SPT,), jnp.int32),
        pltpu.VMEM((HTOT,), jnp.float32),
        pltpu.VMEM((NTILES, 256), jnp.float32),
        pltpu.VMEM((NTILES, 256), jnp.float32),
        pltpu.VMEM((NTILES, 256), jnp.float32),
        pltpu.VMEM((NTILES, 32), jnp.float32),
        pltpu.VMEM((256,), jnp.float32),
        pltpu.VMEM((256,), jnp.float32),
        pltpu.VMEM((256,), jnp.float32),
        pltpu.VMEM((32,), jnp.float32),
        pltpu.VMEM((NB, L), jnp.float32),
        pltpu.VMEM((NB, L), jnp.float32),
        pltpu.VMEM((NB, L), jnp.float32),
        pltpu.VMEM((L,), jnp.float32),
        pltpu.VMEM((L,), jnp.float32),
        pltpu.VMEM((L,), jnp.float32),
        pltpu.VMEM_SHARED((NTILES, HTOT), jnp.float32),
        pltpu.VMEM_SHARED((NB, C), jnp.float32),
        pltpu.VMEM_SHARED((NB, C), jnp.float32),
        pltpu.VMEM_SHARED((NB, C), jnp.float32),
        pltpu.VMEM_SHARED((512,), jnp.float32),
        pltpu.VMEM_SHARED((NTILES, L), jnp.float32),
        pltpu.VMEM_SHARED((NTILES, L), jnp.float32),
        pltpu.SemaphoreType.DMA,
        ],
    )


def kernel(y_pred, y_true):
    conf, pred = _tc_stage(y_pred)
    out = _sc_stage_fn()(conf, pred, y_true)
    return out[0:1]


# speedup vs baseline: 1.1084x; 1.0229x over previous
import functools

import jax
import jax.numpy as jnp
import numpy as np
from jax import lax
from jax.experimental import pallas as pl
from jax.experimental.pallas import tpu as pltpu
from jax.experimental.pallas import tpu_sc as plsc

N = 16384
C = 256
N_BINS = 15
NB = 16
HC = C * NB
OFF_CNT = 0
OFF_CNF = HC
OFF_ACC = 2 * HC
OFF_YT = 3 * HC
HTOT = 3 * HC + 512

NTILES = 16
SPT = N // NTILES
L = 16

_LOWERS = np.linspace(0.0, 1.0, N_BINS + 1)[:-1].astype(np.float32)


_BLK = 4096


def _tc_body(yp_ref, conf_ref, pred_ref):
    x = yp_ref[...]
    m = jnp.max(x, axis=1, keepdims=True)
    e = jnp.exp(x - m)
    ones = jnp.ones((C, 128), jnp.float32)
    s = lax.dot_general(e, ones, (((1,), (0,)), ((), ())),
                        preferred_element_type=jnp.float32)[:, :1]
    conf = 1.0 / s
    iota = lax.broadcasted_iota(jnp.int32, x.shape, 1)
    pred = jnp.min(jnp.where(x == m, iota, jnp.int32(2**30)),
                   axis=1, keepdims=True).astype(jnp.float32)
    cp = jnp.concatenate([conf, pred], axis=1)
    r0 = lax.broadcasted_iota(jnp.int32, (128, 128), 0)
    r1 = lax.broadcasted_iota(jnp.int32, (128, 128), 1)
    eye = (r0 == r1).astype(jnp.float32)
    for k in range(_BLK // 128):
        chunk = cp[k * 128:(k + 1) * 128, :]
        outc = lax.dot_general(chunk, eye, (((0,), (0,)), ((), ())),
                               preferred_element_type=jnp.float32)
        conf_ref[0, 0, pl.ds(k * 128, 128)] = outc[0]
        pred_ref[0, 0, pl.ds(k * 128, 128)] = outc[1]


def _tc_stage(y_pred):
    g = N // _BLK
    conf, pred = pl.pallas_call(
        _tc_body,
        grid=(g,),
        in_specs=[
            pl.BlockSpec((_BLK, C), lambda i: (i, 0)),
        ],
        out_specs=[
            pl.BlockSpec((1, 1, _BLK), lambda i: (i, 0, 0)),
            pl.BlockSpec((1, 1, _BLK), lambda i: (i, 0, 0)),
        ],
        out_shape=[
            jax.ShapeDtypeStruct((g, 1, _BLK), jnp.float32),
            jax.ShapeDtypeStruct((g, 1, _BLK), jnp.float32),
        ],
        compiler_params=pltpu.CompilerParams(skip_device_barrier=True),
    )(y_pred)
    return conf.reshape(N), pred.reshape(N)


def _sc_body(cnf_hbm, prd_hbm, yt_hbm, out_hbm,
             cnf_v, prd_v, yt_v, hist_v,
             blkc, blkf, blka, blkyt, redc, redf, reda, redyt,
             cnt16, cnf16, acc16, yt16, stage16, stage16b,
             sh_all, sh_cnt, sh_cnf, sh_acc, sh_yt, sh_p, sh_c, sem):
    cid = lax.axis_index("c")
    wid = lax.axis_index("s")

    def batch(*copies):
        handles = [pltpu.async_copy(s, d, sem) for s, d in copies]
        for h in handles:
            h.wait()

    @pl.when(cid == 0)
    def _():
        ones = jnp.full((L,), 1.0, jnp.float32)

        base = wid * SPT
        h1 = pltpu.async_copy(cnf_hbm.at[pl.ds(base, SPT)], cnf_v, sem)
        h2 = pltpu.async_copy(prd_hbm.at[pl.ds(base, SPT)], prd_v, sem)
        h3 = pltpu.async_copy(yt_hbm.at[pl.ds(base, SPT)], yt_v, sem)
        zero = jnp.zeros((L,), jnp.float32)

        def zero_body(i, _):
            for u in range(4):
                hist_v[pl.ds(i * (4 * L) + u * L, L)] = zero
            return 0

        lax.fori_loop(0, HTOT // L // 4, zero_body, 0)
        h1.wait()
        h2.wait()
        h3.wait()
        def scatter_body(i, _):
            for u in range(4):
                sl = pl.ds(i * (4 * L) + u * L, L)
                c16 = cnf_v[sl]
                p16 = prd_v[sl].astype(jnp.int32)
                t16 = yt_v[sl]
                a16 = (p16 == t16).astype(jnp.float32)
                t = c16 * jnp.float32(N_BINS)
                f = t.astype(jnp.int32)
                b16 = f - (t == f.astype(jnp.float32)).astype(jnp.int32)
                k16 = b16 * C + p16
                plsc.addupdate_scatter(hist_v, [k16], ones)
                plsc.addupdate_scatter(hist_v, [k16 + OFF_CNF], c16)
                plsc.addupdate_scatter(hist_v, [k16 + OFF_ACC], a16)
                plsc.addupdate_scatter(hist_v, [t16 + OFF_YT], ones)
            return 0

        lax.fori_loop(0, SPT // L // 4, scatter_body, 0)

        pltpu.sync_copy(hist_v, sh_all.at[wid])
        plsc.subcore_barrier()

        batch((sh_all.at[:, pl.ds(OFF_CNT + wid * 256, 256)], blkc),
              (sh_all.at[:, pl.ds(OFF_CNF + wid * 256, 256)], blkf),
              (sh_all.at[:, pl.ds(OFF_ACC + wid * 256, 256)], blka),
              (sh_all.at[:, pl.ds(OFF_YT + wid * 32, 32)], blkyt))

        def reduce_region(blk, red, width):
            def body(j, _):
                sl = pl.ds(j * L, L)
                acc = blk[0, sl]
                for t in range(1, NTILES):
                    acc = acc + blk[t, sl]
                red[sl] = acc
                return 0
            lax.fori_loop(0, width // L, body, 0)

        reduce_region(blkc, redc, 256)
        reduce_region(blkf, redf, 256)
        reduce_region(blka, reda, 256)
        reduce_region(blkyt, redyt, 32)
        batch((redc, sh_cnt.at[wid]),
              (redf, sh_cnf.at[wid]),
              (reda, sh_acc.at[wid]),
              (redyt, sh_yt.at[pl.ds(wid * 32, 32)]))
        plsc.subcore_barrier()

        cls0 = wid * L
        batch((sh_cnt.at[:, pl.ds(cls0, L)], cnt16),
              (sh_cnf.at[:, pl.ds(cls0, L)], cnf16),
              (sh_acc.at[:, pl.ds(cls0, L)], acc16),
              (sh_yt.at[pl.ds(cls0, L)], yt16))
        n_c = jnp.zeros((L,), jnp.float32)
        esum = jnp.zeros((L,), jnp.float32)
        for b in range(NB):
            kv = cnt16[b]
            cv = cnf16[b]
            av = acc16[b]
            n_c = n_c + kv
            esum = esum + jnp.where(kv > 0.0, jnp.abs(cv - av), 0.0)
        ece = esum / jnp.maximum(n_c, 1.0)
        pv = (yt16[...] > 0.0).astype(jnp.float32)
        stage16[...] = jnp.broadcast_to(jnp.sum(pv * ece), (L,))
        stage16b[...] = jnp.broadcast_to(jnp.sum(pv), (L,))
        batch((stage16, sh_p.at[wid]),
              (stage16b, sh_c.at[wid]))
        plsc.subcore_barrier()

        @pl.when(wid == 0)
        def _():
            batch((sh_p, cnt16),
                  (sh_c, cnf16))
            accp = cnt16[0]
            accc = cnf16[0]
            for t in range(1, NTILES):
                accp = accp + cnt16[t]
                accc = accc + cnf16[t]
            stage16[...] = accp / accc
            pltpu.sync_copy(stage16, out_hbm)


@functools.lru_cache(maxsize=1)
def _sc_stage_fn():
    return pl.kernel(
        _sc_body,
        mesh=plsc.VectorSubcoreMesh(core_axis_name="c", subcore_axis_name="s", num_cores=1),
        compiler_params=pltpu.CompilerParams(
            use_tc_tiling_on_sc=False, needs_layout_passes=False,
            skip_device_barrier=True),
        out_type=jax.ShapeDtypeStruct((L,), jnp.float32),
        scratch_types=[
        pltpu.VMEM((SPT,), jnp.float32),
        pltpu.VMEM((SPT,), jnp.float32),
        pltpu.VMEM((SPT,), jnp.int32),
        pltpu.VMEM((HTOT,), jnp.float32),
        pltpu.VMEM((NTILES, 256), jnp.float32),
        pltpu.VMEM((NTILES, 256), jnp.float32),
        pltpu.VMEM((NTILES, 256), jnp.float32),
        pltpu.VMEM((NTILES, 32), jnp.float32),
        pltpu.VMEM((256,), jnp.float32),
        pltpu.VMEM((256,), jnp.float32),
        pltpu.VMEM((256,), jnp.float32),
        pltpu.VMEM((32,), jnp.float32),
        pltpu.VMEM((NB, L), jnp.float32),
        pltpu.VMEM((NB, L), jnp.float32),
        pltpu.VMEM((NB, L), jnp.float32),
        pltpu.VMEM((L,), jnp.float32),
        pltpu.VMEM((L,), jnp.float32),
        pltpu.VMEM((L,), jnp.float32),
        pltpu.VMEM_SHARED((NTILES, HTOT), jnp.float32),
        pltpu.VMEM_SHARED((NB, C), jnp.float32),
        pltpu.VMEM_SHARED((NB, C), jnp.float32),
        pltpu.VMEM_SHARED((NB, C), jnp.float32),
        pltpu.VMEM_SHARED((512,), jnp.float32),
        pltpu.VMEM_SHARED((NTILES, L), jnp.float32),
        pltpu.VMEM_SHARED((NTILES, L), jnp.float32),
        pltpu.SemaphoreType.DMA,
        ],
    )


def kernel(y_pred, y_true):
    conf, pred = _tc_stage(y_pred)
    out = _sc_stage_fn()(conf, pred, y_true)
    return out[0:1]
